# baseline (device time: 116367 ns/iter reference)
import jax
import jax.numpy as jnp
import numpy as np
from jax import lax
from jax.experimental import pallas as pl
from jax.experimental.pallas import tpu as pltpu

N_DEV = 32
N_SEG = 2
WIRE_DTYPE = jnp.bfloat16

_XY_TO_P = {(0, 0): 0, (1, 0): 1, (1, 1): 2, (0, 1): 3,
            (0, 2): 4, (1, 2): 5, (1, 3): 6, (0, 3): 7}


def _build_cycle() -> np.ndarray:
    path = []
    for y in range(4):
        zs = range(4) if y % 2 == 0 else range(3, -1, -1)
        path.extend((y, z) for z in zs)
    cycle = [(0, y, z) for (y, z) in path]
    cycle += [(1, y, z) for (y, z) in reversed(path)]
    return np.array([z * 8 + _XY_TO_P[(x, y)] for (x, y, z) in cycle],
                    dtype=np.int32)


PERM = _build_cycle()
PERM_INV = np.zeros(N_DEV, np.int32)
PERM_INV[PERM] = np.arange(N_DEV, dtype=np.int32)


def kernel(x, w_mat, scale_x, scale_w):
    k_global, _ = x.shape
    _, n = w_mat.shape
    m_per = k_global // N_DEV
    nh = n // 2
    nq = nh // N_SEG
    n_lanes = 2 * N_SEG

    perm = jnp.asarray(PERM)
    pos = jnp.asarray(PERM_INV)[lax.axis_index("i")]
    left = perm[(pos - 1) % N_DEV].reshape(1)
    right = perm[(pos + 1) % N_DEV].reshape(1)
    sidx = jnp.arange(N_DEV, dtype=jnp.int32)
    cs_cw = perm[(pos - 1 - sidx) % N_DEV]
    cs_ccw = perm[(pos + 1 + sidx) % N_DEV]

    def body(x_ref, w_ref, sx_ref, sw_ref, left_ref, right_ref,
             cs_cw_ref, cs_ccw_ref, out_ref, *scratch):
        lanes = [scratch[5 * li:5 * li + 5] for li in range(n_lanes)]

        lft = left_ref[0]
        rgt = right_ref[0]
        dst = [rgt] * N_SEG + [lft] * N_SEG
        ups = [lft] * N_SEG + [rgt] * N_SEG

        barrier_sem = pltpu.get_barrier_semaphore()
        for nbr in (lft, rgt):
            pl.semaphore_signal(barrier_sem, inc=1, device_id=(nbr,),
                                device_id_type=pl.DeviceIdType.MESH)
        pl.semaphore_wait(barrier_sem, 2)

        wl = w_ref[:, :nh].astype(jnp.bfloat16)
        wr = w_ref[:, nh:].astype(jnp.bfloat16)
        scale = sx_ref[0] * sw_ref[0]

        def partials(s):
            c1 = cs_cw_ref[s]
            c2 = cs_ccw_ref[s]
            x1 = x_ref[pl.ds(c1 * m_per, m_per), :].astype(jnp.bfloat16)
            x2 = x_ref[pl.ds(c2 * m_per, m_per), :].astype(jnp.bfloat16)
            dims = (((1,), (0,)), ((), ()))
            p_cw = lax.dot_general(x1, wl, dims,
                                   preferred_element_type=jnp.float32)
            p_ccw = lax.dot_general(x2, wr, dims,
                                    preferred_element_type=jnp.float32)
            ps = [p_cw[:, g * nq:(g + 1) * nq] for g in range(N_SEG)]
            ps += [p_ccw[:, g * nq:(g + 1) * nq] for g in range(N_SEG)]
            if s < N_DEV - 1:
                ps = [p.astype(WIRE_DTYPE) for p in ps]
            return ps

        ps = partials(0)
        prev = [None] * n_lanes
        prev2 = [None] * n_lanes
        for s in range(N_DEV):
            for li in range(n_lanes):
                send_buf, recv_buf, send_sems, recv_sems, credit = \
                    lanes[li]
                if s == 0:
                    acc = ps[li]
                else:
                    prev[li].wait_recv()
                    recv = recv_buf[(s - 1) % 2, :, :]
                    if s == N_DEV - 1:
                        recv = recv.astype(jnp.float32)
                    acc = ps[li] + recv
                    if s <= N_DEV - 3:
                        pl.semaphore_signal(
                            credit, inc=1, device_id=(ups[li],),
                            device_id_type=pl.DeviceIdType.MESH)
                if s < N_DEV - 1:
                    if s >= 2:
                        prev2[li].wait_send()
                    send_buf[s % 2, :, :] = acc
                    if s >= 2:
                        pl.semaphore_wait(credit, 1)
                    rdma = pltpu.make_async_remote_copy(
                        src_ref=send_buf.at[s % 2],
                        dst_ref=recv_buf.at[s % 2],
                        send_sem=send_sems.at[s % 2],
                        recv_sem=recv_sems.at[s % 2],
                        device_id=(dst[li],),
                        device_id_type=pl.DeviceIdType.MESH,
                    )
                    rdma.start()
                    prev2[li] = prev[li]
                    prev[li] = rdma
                else:
                    out_ref[:, li * nq:(li + 1) * nq] = acc * scale
            if s < N_DEV - 1:
                ps = partials(s + 1)
        for li in range(n_lanes):
            prev2[li].wait_send()
            prev[li].wait_send()

    lane_scratch = []
    for _ in range(n_lanes):
        lane_scratch += [
            pltpu.VMEM((2, m_per, nq), WIRE_DTYPE),
            pltpu.VMEM((2, m_per, nq), WIRE_DTYPE),
            pltpu.SemaphoreType.DMA((2,)),
            pltpu.SemaphoreType.DMA((2,)),
            pltpu.SemaphoreType.REGULAR,
        ]

    return pl.pallas_call(
        body,
        out_shape=jax.ShapeDtypeStruct((m_per, n), jnp.float32),
        in_specs=[pl.BlockSpec(memory_space=pltpu.VMEM)] * 4
        + [pl.BlockSpec(memory_space=pltpu.SMEM)] * 4,
        out_specs=pl.BlockSpec(memory_space=pltpu.VMEM),
        scratch_shapes=lane_scratch,
        compiler_params=pltpu.CompilerParams(collective_id=0),
    )(x, w_mat, scale_x, scale_w, left, right, cs_cw, cs_ccw)
